# baseline (speedup 1.0000x reference)
import functools

import jax
import jax.numpy as jnp
from jax import lax
from jax.experimental import pallas as pl
from jax.experimental.pallas import tpu as pltpu, tpu_sc as plsc

TIME_STEPS = 1000
EMBED_DIM = 512
BATCH = 1024
_LANE = 128
_CHUNKS = EMBED_DIM // _LANE

_NUM_SUBCORES = 16
_B_PER_W = BATCH // _NUM_SUBCORES
_BLK = 32
_C_BLK = _BLK * _CHUNKS

_mesh = plsc.VectorSubcoreMesh(
    core_axis_name="c", subcore_axis_name="s", num_cores=1)


@functools.partial(
    pl.kernel,
    mesh=_mesh,
    out_type=jax.ShapeDtypeStruct((BATCH * _CHUNKS, _LANE), jnp.float32),
    compiler_params=pltpu.CompilerParams(
        use_tc_tiling_on_sc=False, needs_layout_passes=False),
    scratch_types=[
        pltpu.VMEM((_B_PER_W,), jnp.int32),
        pltpu.VMEM((_C_BLK,), jnp.int32),
        pltpu.VMEM((_C_BLK,), jnp.int32),
        pltpu.VMEM((_C_BLK, _LANE), jnp.float32),
        pltpu.VMEM((_C_BLK, _LANE), jnp.float32),
        pltpu.SemaphoreType.DMA,
        pltpu.SemaphoreType.DMA,
        pltpu.SemaphoreType.DMA,
        pltpu.SemaphoreType.DMA,
    ],
)
def _gather_rows(table_hbm, idx_hbm, out_hbm, idx_v, idxA, idxB, rowsA, rowsB,
                 gA, gB, sA, sB):
    sid = lax.axis_index("s")
    base = sid * _B_PER_W
    pltpu.sync_copy(idx_hbm.at[pl.ds(base, _B_PER_W)], idx_v)

    def fill(idx4, off):
        for k in range(_BLK // 16):
            tv = idx_v[pl.ds(off + k * 16, 16)]
            pb = ((tv >> 3) << 5) | (tv & 7)
            pos = lax.iota(jnp.int32, 16) * _CHUNKS + k * 16 * _CHUNKS
            for c in range(_CHUNKS):
                plsc.store_scatter(idx4, [pos + c], pb + c * 8)

    fill(idxA, 0)
    cA = pltpu.async_copy(table_hbm.at[idxA], rowsA, gA)
    fill(idxB, _BLK)
    cB = pltpu.async_copy(table_hbm.at[idxB], rowsB, gB)
    cA.wait()
    wA = pltpu.async_copy(rowsA, out_hbm.at[pl.ds(base * _CHUNKS, _C_BLK)], sA)
    cB.wait()
    wB = pltpu.async_copy(
        rowsB, out_hbm.at[pl.ds(base * _CHUNKS + _C_BLK, _C_BLK)], sB)
    wA.wait()
    wB.wait()


def kernel(x, t, embeddings):
    table4 = (
        embeddings.reshape(TIME_STEPS // 8, 8, _CHUNKS, _LANE)
        .transpose(0, 2, 1, 3)
        .reshape(TIME_STEPS * _CHUNKS, _LANE)
    )
    out = _gather_rows(table4, t.astype(jnp.int32))
    return out.reshape(BATCH, EMBED_DIM, 1, 1)
